# baseline (speedup 1.0000x reference)
import functools

import jax
import jax.numpy as jnp
from jax import lax
from jax.experimental import pallas as pl
from jax.experimental.pallas import tpu as pltpu
from jax.experimental.pallas import tpu_sc as plsc

_NC = 2
_NS = 16
_NW = _NC * _NS
_L = 16

_ROWS = 8192
_COLS = 2048
_ROWS_PER_W = _ROWS // _NW
_R = 16
_CHUNKS = _ROWS_PER_W // _R
_JBLK = _COLS // _L


def _permute_body(x_hbm, idx_hbm, out_hbm, idx_v, in_v, out_v):
    wid = lax.axis_index("s") * _NC + lax.axis_index("c")
    base = wid * _ROWS_PER_W * _COLS

    pltpu.sync_copy(idx_hbm, idx_v)

    def chunk_body(c, carry):
        off = base + c * (_R * _COLS)
        pltpu.sync_copy(x_hbm.at[pl.ds(off, _R * _COLS)], in_v)

        def col_body(j, carry2):
            iv = idx_v[pl.ds(j * _L, _L)]
            joff = j * _L
            for r in range(_R):
                vals = plsc.load_gather(in_v, [iv + r * _COLS])
                out_v[pl.ds(r * _COLS + joff, _L)] = vals
            return carry2

        lax.fori_loop(0, _JBLK, col_body, 0, unroll=False)
        pltpu.sync_copy(out_v, out_hbm.at[pl.ds(off, _R * _COLS)])
        return carry

    lax.fori_loop(0, _CHUNKS, chunk_body, 0, unroll=False)


@functools.partial(jax.jit, static_argnames=())
def kernel(x, idxs):
    x_flat = x.reshape(_ROWS * _COLS)
    mesh = plsc.VectorSubcoreMesh(
        core_axis_name="c", subcore_axis_name="s", num_cores=_NC,
        num_subcores=_NS)
    out_flat = pl.kernel(
        _permute_body,
        out_type=jax.ShapeDtypeStruct((_ROWS * _COLS,), jnp.float32),
        mesh=mesh,
        scratch_types=[
            pltpu.VMEM((_COLS,), jnp.int32),
            pltpu.VMEM((_R * _COLS,), jnp.float32),
            pltpu.VMEM((_R * _COLS,), jnp.float32),
        ],
        compiler_params=pltpu.CompilerParams(needs_layout_passes=False),
    )(x_flat, idxs)
    return out_flat.reshape(x.shape)


# speedup vs baseline: 1.7910x; 1.7910x over previous
import functools

import jax
import jax.numpy as jnp
from jax import lax
from jax.experimental import pallas as pl
from jax.experimental.pallas import tpu as pltpu
from jax.experimental.pallas import tpu_sc as plsc

_NC = 2
_NS = 16
_NW = _NC * _NS
_L = 16

_ROWS = 8192
_COLS = 2048
_ROWS_PER_W = _ROWS // _NW
_R = 8
_CHUNK = _R * _COLS
_NCHUNK = _ROWS_PER_W // _R
_G = _NCHUNK // 2
_JBLK = _COLS // _L


def _permute_body(x_hbm, idx_hbm, out_hbm, idx_v,
                  in0, in1, out0, out1, sin0, sin1, sout0, sout1):
    wid = lax.axis_index("s") * _NC + lax.axis_index("c")
    elem_base = wid * _ROWS_PER_W * _COLS

    ins = (in0, in1)
    outs = (out0, out1)
    sins = (sin0, sin1)
    souts = (sout0, sout1)

    pltpu.sync_copy(idx_hbm, idx_v)

    def in_start(c, b):
        pltpu.async_copy(x_hbm.at[pl.ds(elem_base + c * _CHUNK, _CHUNK)],
                         ins[b], sins[b])

    def in_wait(b):
        pltpu.make_async_copy(x_hbm.at[pl.ds(elem_base, _CHUNK)], ins[b],
                              sins[b]).wait()

    def out_start(c, b):
        pltpu.async_copy(outs[b],
                         out_hbm.at[pl.ds(elem_base + c * _CHUNK, _CHUNK)],
                         souts[b])

    def out_wait(b):
        pltpu.make_async_copy(outs[b], out_hbm.at[pl.ds(elem_base, _CHUNK)],
                              souts[b]).wait()

    def compute(b):
        in_b = ins[b]
        out_b = outs[b]

        @plsc.parallel_loop(0, _JBLK, 1, unroll=4)
        def _col(j):
            joff = j * _L
            iv = idx_v[pl.ds(joff, _L)]
            for r in range(_R):
                out_b[pl.ds(r * _COLS + joff, _L)] = plsc.load_gather(
                    in_b, [iv + r * _COLS])

    in_start(0, 0)
    in_start(1, 1)
    for b in (0, 1):
        in_wait(b)
        compute(b)
        out_start(b, b)
        in_start(b + 2, b)

    def g_body(g, carry):
        for b in (0, 1):
            c = 2 * g + b
            in_wait(b)
            out_wait(b)
            compute(b)
            out_start(c, b)
            in_start(c + 2, b)
        return carry

    lax.fori_loop(1, _G - 1, g_body, 0, unroll=False)

    for b in (0, 1):
        c = 2 * (_G - 1) + b
        in_wait(b)
        out_wait(b)
        compute(b)
        out_start(c, b)
    out_wait(0)
    out_wait(1)


@functools.partial(jax.jit, static_argnames=())
def kernel(x, idxs):
    x1d = x.reshape(_ROWS * _COLS)
    mesh = plsc.VectorSubcoreMesh(
        core_axis_name="c", subcore_axis_name="s", num_cores=_NC,
        num_subcores=_NS)
    out1d = pl.kernel(
        _permute_body,
        out_type=jax.ShapeDtypeStruct((_ROWS * _COLS,), jnp.float32),
        mesh=mesh,
        scratch_types=[
            pltpu.VMEM((_COLS,), jnp.int32),
            pltpu.VMEM((_CHUNK,), jnp.float32),
            pltpu.VMEM((_CHUNK,), jnp.float32),
            pltpu.VMEM((_CHUNK,), jnp.float32),
            pltpu.VMEM((_CHUNK,), jnp.float32),
            pltpu.SemaphoreType.DMA,
            pltpu.SemaphoreType.DMA,
            pltpu.SemaphoreType.DMA,
            pltpu.SemaphoreType.DMA,
        ],
        compiler_params=pltpu.CompilerParams(needs_layout_passes=False),
    )(x1d, idxs)
    return out1d.reshape(x.shape)


# speedup vs baseline: 4.8095x; 2.6854x over previous
import functools

import jax
import jax.numpy as jnp
from jax import lax
from jax.experimental import pallas as pl
from jax.experimental.pallas import tpu as pltpu
from jax.experimental.pallas import tpu_sc as plsc

_NC = 2
_NS = 16
_NW = _NC * _NS
_L = 16

_ROWS = 8192
_COLS = 2048
_ROWS_PER_W = _ROWS // _NW
_R = 8
_CHUNK = _R * _COLS
_NCHUNK = _ROWS_PER_W // _R
_G = _NCHUNK // 2
_JBLK = _COLS // _L


def _permute_body(x_hbm, idx_hbm, out_hbm, idx_v,
                  in0, in1, out0, out1, sin0, sin1, sout0, sout1):
    wid = lax.axis_index("s") * _NC + lax.axis_index("c")
    row_base = wid * _ROWS_PER_W

    ins = (in0, in1)
    outs = (out0, out1)
    sins = (sin0, sin1)
    souts = (sout0, sout1)

    pltpu.sync_copy(idx_hbm, idx_v)

    def in_start(c, b):
        pltpu.async_copy(x_hbm.at[pl.ds(row_base + c * _R, _R)], ins[b],
                         sins[b])

    def in_wait(b):
        pltpu.make_async_copy(x_hbm.at[pl.ds(row_base, _R)], ins[b],
                              sins[b]).wait()

    def out_start(c, b):
        pltpu.async_copy(outs[b], out_hbm.at[pl.ds(row_base + c * _R, _R)],
                         souts[b])

    def out_wait(b):
        pltpu.make_async_copy(outs[b], out_hbm.at[pl.ds(row_base, _R)],
                              souts[b]).wait()

    def compute(b):
        in_b = ins[b]
        out_b = outs[b]

        @plsc.parallel_loop(0, _JBLK, 1, unroll=4)
        def _col(j):
            joff = j * _L
            iv = idx_v[pl.ds(joff, _L)]
            for r in range(_R):
                rv = jnp.full((_L,), r, jnp.int32)
                out_b[r, pl.ds(joff, _L)] = plsc.load_gather(in_b, [rv, iv])

    in_start(0, 0)
    in_start(1, 1)
    for b in (0, 1):
        in_wait(b)
        compute(b)
        out_start(b, b)
        in_start(b + 2, b)

    def g_body(g, carry):
        for b in (0, 1):
            c = 2 * g + b
            in_wait(b)
            out_wait(b)
            compute(b)
            out_start(c, b)
            in_start(c + 2, b)
        return carry

    lax.fori_loop(1, _G - 1, g_body, 0, unroll=False)

    for b in (0, 1):
        c = 2 * (_G - 1) + b
        in_wait(b)
        out_wait(b)
        compute(b)
        out_start(c, b)
    out_wait(0)
    out_wait(1)


@functools.partial(jax.jit, static_argnames=())
def kernel(x, idxs):
    x2d = x.reshape(_ROWS, _COLS)
    mesh = plsc.VectorSubcoreMesh(
        core_axis_name="c", subcore_axis_name="s", num_cores=_NC,
        num_subcores=_NS)
    out2d = pl.kernel(
        _permute_body,
        out_type=jax.ShapeDtypeStruct((_ROWS, _COLS), jnp.float32),
        mesh=mesh,
        scratch_types=[
            pltpu.VMEM((_COLS,), jnp.int32),
            pltpu.VMEM((_R, _COLS), jnp.float32),
            pltpu.VMEM((_R, _COLS), jnp.float32),
            pltpu.VMEM((_R, _COLS), jnp.float32),
            pltpu.VMEM((_R, _COLS), jnp.float32),
            pltpu.SemaphoreType.DMA,
            pltpu.SemaphoreType.DMA,
            pltpu.SemaphoreType.DMA,
            pltpu.SemaphoreType.DMA,
        ],
        compiler_params=pltpu.CompilerParams(needs_layout_passes=False),
    )(x2d, idxs)
    return out2d.reshape(x.shape)
